# baseline (speedup 1.0000x reference)
import functools

import jax
import jax.numpy as jnp
from jax.experimental import pallas as pl
from jax.experimental.pallas import tpu as pltpu

_VT = 1024
_NC = 12
_KOUT = 10
_LANES = 128
_NEG = float("-inf")
_IMAX = jnp.iinfo(jnp.int32).max


def _topk_kernel(state_ref, emb_ref, idx_ref, prob_ref,
                 cv_ref, ci_ref, s_ref, *, V, B):
    i = pl.program_id(1)
    nt = pl.num_programs(1)

    @pl.when(i == 0)
    def _init():
        cv_ref[...] = jnp.full((B, _LANES), _NEG, jnp.float32)
        ci_ref[...] = jnp.full((B, _LANES), _IMAX, jnp.int32)
        s_ref[...] = jnp.zeros((B, _LANES), jnp.float32)

    logits = jax.lax.dot_general(
        state_ref[...], emb_ref[...], (((1,), (1,)), ((), ())),
        preferred_element_type=jnp.float32)
    liota = jax.lax.broadcasted_iota(jnp.int32, logits.shape, 1)
    logits = jnp.where(liota < V - i * _VT, logits, _NEG)

    lane = jax.lax.broadcasted_iota(jnp.int32, (B, _LANES), 1)
    m_old = cv_ref[...][:, :1]
    kth = cv_ref[...][:, _NC - 1:_NC]

    ngrp = _VT // _LANES
    cm1 = logits[:, :_LANES]
    cm2 = jnp.full((B, _LANES), _NEG, jnp.float32)
    cs1 = jnp.zeros((B, _LANES), jnp.int32)
    cs2 = jnp.zeros((B, _LANES), jnp.int32)
    cacc = (cm1 > kth).astype(jnp.int32)
    for g in range(1, ngrp):
        seg = logits[:, g * _LANES:(g + 1) * _LANES]
        b1 = seg > cm1
        b2 = seg > cm2
        cm2 = jnp.where(b1, cm1, jnp.where(b2, seg, cm2))
        cs2 = jnp.where(b1, cs1, jnp.where(b2, g, cs2))
        cm1 = jnp.where(b1, seg, cm1)
        cs1 = jnp.where(b1, g, cs1)
        cacc = cacc + (seg > kth).astype(jnp.int32)
    cnt_row = jnp.sum(cacc, axis=1, keepdims=True)
    covered = (jnp.sum((cm1 > kth).astype(jnp.int32), axis=1, keepdims=True)
               + jnp.sum((cm2 > kth).astype(jnp.int32), axis=1, keepdims=True))
    clash = jnp.max(jnp.where(cnt_row > covered, 1, 0)) > 0
    trip = jnp.minimum(jnp.max(cnt_row), _NC)
    pool = jnp.concatenate([cm1, cm2], axis=1)
    pool_pi = jnp.concatenate([cs1 * _LANES + lane, cs2 * _LANES + lane],
                              axis=1)

    def _insert(cv, ci, mx, sel):
        pos = jnp.sum(((cv > mx) | ((cv == mx) & (ci < sel))).astype(jnp.int32),
                      axis=1, keepdims=True)
        sv = jnp.roll(cv, 1, axis=1)
        si = jnp.roll(ci, 1, axis=1)
        cv = jnp.where(lane < pos, cv, jnp.where(lane == pos, mx, sv))
        ci = jnp.where(lane < pos, ci, jnp.where(lane == pos, sel, si))
        cv = jnp.where(lane >= _NC, _NEG, cv)
        ci = jnp.where(lane >= _NC, _IMAX, ci)
        return cv, ci

    @pl.when(clash)
    def _merge_full():
        def _body(_, carry):
            tv, cv, ci = carry
            mx = jnp.max(tv, axis=1, keepdims=True)
            sel = jnp.min(jnp.where(tv == mx, liota, _IMAX), axis=1,
                          keepdims=True)
            tv = jnp.where((tv == mx) & (liota == sel), _NEG, tv)
            cv, ci = _insert(cv, ci, mx, sel + i * _VT)
            return tv, cv, ci

        _, nv, ni = jax.lax.fori_loop(
            0, trip, _body, (logits, cv_ref[...], ci_ref[...]))
        cv_ref[...] = nv
        ci_ref[...] = ni

    @pl.when(jnp.logical_not(clash))
    def _merge_proxy():
        def _body(_, carry):
            pm, cv, ci = carry
            mx = jnp.max(pm, axis=1, keepdims=True)
            pi = jnp.min(jnp.where(pm == mx, pool_pi, _IMAX), axis=1,
                         keepdims=True)
            pm = jnp.where(pool_pi == pi, _NEG, pm)
            cv, ci = _insert(cv, ci, mx, i * _VT + pi)
            return pm, cv, ci

        _, nv, ni = jax.lax.fori_loop(
            0, trip, _body, (pool, cv_ref[...], ci_ref[...]))
        cv_ref[...] = nv
        ci_ref[...] = ni

    m_new = jnp.maximum(m_old, jnp.max(cm1, axis=1, keepdims=True))
    s_new = (s_ref[...][:, :1] * jnp.exp(m_old - m_new)
             + jnp.sum(jnp.exp(logits - m_new), axis=1, keepdims=True))
    s_ref[...] = jnp.broadcast_to(s_new, (B, _LANES))

    new_v = cv_ref[...]
    new_i = ci_ref[...]

    @pl.when(i == nt - 1)
    def _final():
        pv = jnp.exp(new_v - m_new) / s_new
        pidx = new_i
        ov = jnp.zeros((B, _LANES), jnp.float32)
        oi = jnp.zeros((B, _LANES), jnp.int32)
        for k in range(_KOUT):
            mx = jnp.max(pv, axis=1, keepdims=True)
            sel = jnp.min(jnp.where(pv == mx, pidx, _IMAX), axis=1,
                          keepdims=True)
            ov = jnp.where(lane == k, mx, ov)
            oi = jnp.where(lane == k, sel, oi)
            pv = jnp.where((pv == mx) & (pidx == sel), -1.0, pv)
        idx_ref[...] = oi
        prob_ref[...] = ov


_BB = 2


def kernel(state, item_embeddings, K):
    del K
    B, D = state.shape
    V = item_embeddings.shape[0]
    nt = pl.cdiv(V, _VT)
    bb = B // _BB
    idx128, prob128 = pl.pallas_call(
        functools.partial(_topk_kernel, V=V, B=bb),
        grid=(_BB, nt),
        in_specs=[
            pl.BlockSpec((bb, D), lambda b, i: (b, 0)),
            pl.BlockSpec((_VT, D), lambda b, i: (i, 0)),
        ],
        out_specs=[
            pl.BlockSpec((bb, _LANES), lambda b, i: (b, 0)),
            pl.BlockSpec((bb, _LANES), lambda b, i: (b, 0)),
        ],
        out_shape=[
            jax.ShapeDtypeStruct((B, _LANES), jnp.int32),
            jax.ShapeDtypeStruct((B, _LANES), jnp.float32),
        ],
        scratch_shapes=[
            pltpu.VMEM((bb, _LANES), jnp.float32),
            pltpu.VMEM((bb, _LANES), jnp.int32),
            pltpu.VMEM((bb, _LANES), jnp.float32),
        ],
        compiler_params=pltpu.CompilerParams(
            dimension_semantics=("parallel", "arbitrary")),
    )(state, item_embeddings)
    return (idx128[:, :_KOUT], prob128[:, :_KOUT])


# speedup vs baseline: 1.0195x; 1.0195x over previous
import functools

import jax
import jax.numpy as jnp
from jax.experimental import pallas as pl
from jax.experimental.pallas import tpu as pltpu

_VT = 1024
_NC = 12
_KOUT = 10
_LANES = 128
_NEG = float("-inf")
_IMAX = jnp.iinfo(jnp.int32).max


def _topk_kernel(state_ref, emb_ref, idx_ref, prob_ref,
                 cv_ref, ci_ref, s_ref, *, V, B):
    i = pl.program_id(1)
    nt = pl.num_programs(1)

    @pl.when(i == 0)
    def _init():
        cv_ref[...] = jnp.full((B, _LANES), _NEG, jnp.float32)
        ci_ref[...] = jnp.full((B, _LANES), _IMAX, jnp.int32)
        s_ref[...] = jnp.zeros((B, _LANES), jnp.float32)

    logits = jax.lax.dot_general(
        state_ref[...], emb_ref[...], (((1,), (1,)), ((), ())),
        preferred_element_type=jnp.float32)
    liota = jax.lax.broadcasted_iota(jnp.int32, logits.shape, 1)
    logits = jnp.where(liota < V - i * _VT, logits, _NEG)

    lane = jax.lax.broadcasted_iota(jnp.int32, (B, _LANES), 1)
    m_old = cv_ref[...][:, :1]
    kth = cv_ref[...][:, _NC - 1:_NC]

    ngrp = _VT // _LANES
    cm1 = logits[:, :_LANES]
    cm2 = jnp.full((B, _LANES), _NEG, jnp.float32)
    cs1 = jnp.zeros((B, _LANES), jnp.int32)
    cs2 = jnp.zeros((B, _LANES), jnp.int32)
    cacc = (cm1 > kth).astype(jnp.int32)
    for g in range(1, ngrp):
        seg = logits[:, g * _LANES:(g + 1) * _LANES]
        b1 = seg > cm1
        b2 = seg > cm2
        cm2 = jnp.where(b1, cm1, jnp.where(b2, seg, cm2))
        cs2 = jnp.where(b1, cs1, jnp.where(b2, g, cs2))
        cm1 = jnp.where(b1, seg, cm1)
        cs1 = jnp.where(b1, g, cs1)
        cacc = cacc + (seg > kth).astype(jnp.int32)
    cnt_row = jnp.sum(cacc, axis=1, keepdims=True)
    covered = (jnp.sum((cm1 > kth).astype(jnp.int32), axis=1, keepdims=True)
               + jnp.sum((cm2 > kth).astype(jnp.int32), axis=1, keepdims=True))
    clash = jnp.max(jnp.where(cnt_row > covered, 1, 0)) > 0
    trip = jnp.minimum(jnp.max(cnt_row), _NC)
    pool = jnp.concatenate([cm1, cm2], axis=1)
    pool_pi = jnp.concatenate([cs1 * _LANES + lane, cs2 * _LANES + lane],
                              axis=1)

    def _insert(cv, ci, mx, sel):
        pos = jnp.sum(((cv > mx) | ((cv == mx) & (ci < sel))).astype(jnp.int32),
                      axis=1, keepdims=True)
        sv = jnp.roll(cv, 1, axis=1)
        si = jnp.roll(ci, 1, axis=1)
        cv = jnp.where(lane < pos, cv, jnp.where(lane == pos, mx, sv))
        ci = jnp.where(lane < pos, ci, jnp.where(lane == pos, sel, si))
        cv = jnp.where(lane >= _NC, _NEG, cv)
        ci = jnp.where(lane >= _NC, _IMAX, ci)
        return cv, ci

    @pl.when(clash)
    def _merge_full():
        def _body(_, carry):
            tv, cv, ci = carry
            mx = jnp.max(tv, axis=1, keepdims=True)
            sel = jnp.min(jnp.where(tv == mx, liota, _IMAX), axis=1,
                          keepdims=True)
            tv = jnp.where((tv == mx) & (liota == sel), _NEG, tv)
            cv, ci = _insert(cv, ci, mx, sel + i * _VT)
            return tv, cv, ci

        _, nv, ni = jax.lax.fori_loop(
            0, trip, _body, (logits, cv_ref[...], ci_ref[...]))
        cv_ref[...] = nv
        ci_ref[...] = ni

    @pl.when(jnp.logical_not(clash))
    def _merge_proxy():
        def _body(_, carry):
            pm, cv, ci = carry
            mx = jnp.max(pm, axis=1, keepdims=True)
            pi = jnp.min(jnp.where(pm == mx, pool_pi, _IMAX), axis=1,
                         keepdims=True)
            pm = jnp.where(pool_pi == pi, _NEG, pm)
            cv, ci = _insert(cv, ci, mx, i * _VT + pi)
            return pm, cv, ci

        _, nv, ni = jax.lax.fori_loop(
            0, trip, _body, (pool, cv_ref[...], ci_ref[...]))
        cv_ref[...] = nv
        ci_ref[...] = ni

    new_v = cv_ref[...]
    new_i = ci_ref[...]

    m_new = new_v[:, :1]
    s_new = (s_ref[...][:, :1] * jnp.exp(m_old - m_new)
             + jnp.sum(jnp.exp(logits - m_new), axis=1, keepdims=True))
    s_ref[...] = jnp.broadcast_to(s_new, (B, _LANES))

    @pl.when(i == nt - 1)
    def _final():
        pv = jnp.exp(new_v - m_new) / s_new
        pidx = new_i
        ov = jnp.zeros((B, _LANES), jnp.float32)
        oi = jnp.zeros((B, _LANES), jnp.int32)
        for k in range(_KOUT):
            mx = jnp.max(pv, axis=1, keepdims=True)
            sel = jnp.min(jnp.where(pv == mx, pidx, _IMAX), axis=1,
                          keepdims=True)
            ov = jnp.where(lane == k, mx, ov)
            oi = jnp.where(lane == k, sel, oi)
            pv = jnp.where((pv == mx) & (pidx == sel), -1.0, pv)
        idx_ref[...] = oi
        prob_ref[...] = ov


_BB = 2


def kernel(state, item_embeddings, K):
    del K
    B, D = state.shape
    V = item_embeddings.shape[0]
    nt = pl.cdiv(V, _VT)
    bb = B // _BB
    idx128, prob128 = pl.pallas_call(
        functools.partial(_topk_kernel, V=V, B=bb),
        grid=(_BB, nt),
        in_specs=[
            pl.BlockSpec((bb, D), lambda b, i: (b, 0)),
            pl.BlockSpec((_VT, D), lambda b, i: (i, 0)),
        ],
        out_specs=[
            pl.BlockSpec((bb, _LANES), lambda b, i: (b, 0)),
            pl.BlockSpec((bb, _LANES), lambda b, i: (b, 0)),
        ],
        out_shape=[
            jax.ShapeDtypeStruct((B, _LANES), jnp.int32),
            jax.ShapeDtypeStruct((B, _LANES), jnp.float32),
        ],
        scratch_shapes=[
            pltpu.VMEM((bb, _LANES), jnp.float32),
            pltpu.VMEM((bb, _LANES), jnp.int32),
            pltpu.VMEM((bb, _LANES), jnp.float32),
        ],
        compiler_params=pltpu.CompilerParams(
            dimension_semantics=("parallel", "arbitrary")),
    )(state, item_embeddings)
    return (idx128[:, :_KOUT], prob128[:, :_KOUT])
